# Initial kernel scaffold; baseline (speedup 1.0000x reference)
import jax
import jax.numpy as jnp
from jax.experimental import pallas as pl


def kernel(u2i, i2u, x_user, x_item, w_user, w_item, a_src_u, a_dst_u, a_src_i, a_dst_i):
    raise NotImplementedError("write your pallas kernel here")


# speedup vs baseline: 16.5062x; 16.5062x over previous
import jax
import jax.numpy as jnp
from jax.experimental import pallas as pl
from jax.experimental.pallas import tpu as pltpu

_N = 5000
_NP = 5120
_D = 256
_H = 4
_HP = 8
_SW = 16

_BD = 256
_BS = 512
_NI = _NP // _BD
_NJ = _NP // _BS


def _linear_body(x_ref, w_ref, a_ref, z_ref, s_ref):
    z = jnp.dot(x_ref[...], w_ref[...], preferred_element_type=jnp.float32)
    z_ref[...] = z
    s_ref[...] = jnp.dot(z, a_ref[...], preferred_element_type=jnp.float32)


def _fused_linear(x, w, a_ext):
    return pl.pallas_call(
        _linear_body,
        grid=(_NP // 512,),
        in_specs=[pl.BlockSpec((512, _D), lambda i: (i, 0)),
                  pl.BlockSpec((_D, _D), lambda i: (0, 0)),
                  pl.BlockSpec((_D, _SW), lambda i: (0, 0))],
        out_specs=[pl.BlockSpec((512, _D), lambda i: (i, 0)),
                   pl.BlockSpec((512, _SW), lambda i: (i, 0))],
        out_shape=[jax.ShapeDtypeStruct((_NP, _D), jnp.float32),
                   jax.ShapeDtypeStruct((_NP, _SW), jnp.float32)],
    )(x, w, a_ext)


def _flash_body(c_ref, sd_ref, ss_ref, z_ref, o_ref, m_ref, den_ref, acc_ref):
    j = pl.program_id(1)

    @pl.when(j == 0)
    def _():
        m_ref[...] = jnp.full((_BD, _HP), -jnp.inf, jnp.float32)
        den_ref[...] = jnp.zeros((_BD, _HP), jnp.float32)
        acc_ref[...] = jnp.zeros((_BD, _H * _D), jnp.float32)

    c = c_ref[...]
    mask = c > 0.0
    sd = sd_ref[...]
    ss = ss_ref[...]
    z = z_ref[...]
    m_old = m_ref[...]
    den_old = den_ref[...]

    m_cols, den_cols = [], []
    for h in range(_H):
        e = sd[:, h:h + 1] + ss[h:h + 1, :]
        e = jnp.where(e >= 0.0, e, 0.01 * e)
        e = jnp.where(mask, e, -jnp.inf)
        bm = jnp.max(e, axis=1, keepdims=True)
        mo = m_old[:, h:h + 1]
        mn = jnp.maximum(mo, bm)
        mns = jnp.where(jnp.isfinite(mn), mn, 0.0)
        p = jnp.where(mask, c * jnp.exp(e - mns), 0.0)
        scale = jnp.exp(mo - mns)
        den_cols.append(den_old[:, h:h + 1] * scale
                        + jnp.sum(p, axis=1, keepdims=True))
        acc_ref[:, h * _D:(h + 1) * _D] = (
            acc_ref[:, h * _D:(h + 1) * _D] * scale
            + jnp.dot(p, z, preferred_element_type=jnp.float32))
        m_cols.append(mn)
    m_ref[...] = jnp.concatenate(m_cols + [m_old[:, _H:]], axis=1)
    den_ref[...] = jnp.concatenate(den_cols + [den_old[:, _H:]], axis=1)

    @pl.when(j == _NJ - 1)
    def _():
        den = den_ref[...]
        for h in range(_H):
            o = acc_ref[:, h * _D:(h + 1) * _D] / (den[:, h:h + 1] + 1e-9)
            o_ref[:, h * _D:(h + 1) * _D] = jnp.where(
                o > 0.0, o, jnp.exp(o) - 1.0)


def _flash(c, sd, ss, z):
    return pl.pallas_call(
        _flash_body,
        grid=(_NI, _NJ),
        in_specs=[
            pl.BlockSpec((_BD, _BS), lambda i, j: (i, j)),
            pl.BlockSpec((_BD, _HP), lambda i, j: (i, 0)),
            pl.BlockSpec((_HP, _BS), lambda i, j: (0, j)),
            pl.BlockSpec((_BS, _D), lambda i, j: (j, 0)),
        ],
        out_specs=pl.BlockSpec((_BD, _H * _D), lambda i, j: (i, 0)),
        out_shape=jax.ShapeDtypeStruct((_NP, _H * _D), jnp.float32),
        scratch_shapes=[
            pltpu.VMEM((_BD, _HP), jnp.float32),
            pltpu.VMEM((_BD, _HP), jnp.float32),
            pltpu.VMEM((_BD, _H * _D), jnp.float32),
        ],
        compiler_params=pltpu.CompilerParams(
            dimension_semantics=("parallel", "arbitrary")),
    )(c, sd, ss, z)


def kernel(u2i, i2u, x_user, x_item, w_user, w_item,
           a_src_u, a_dst_u, a_src_i, a_dst_i):
    f32 = jnp.float32
    au_ext = jnp.concatenate(
        [a_src_i.T, a_dst_u.T, jnp.zeros((_D, _SW - 2 * _H), f32)], axis=1)
    ai_ext = jnp.concatenate(
        [a_src_u.T, a_dst_i.T, jnp.zeros((_D, _SW - 2 * _H), f32)], axis=1)
    xu = jnp.zeros((_NP, _D), f32).at[:_N].set(x_user)
    xi = jnp.zeros((_NP, _D), f32).at[:_N].set(x_item)
    z_user, s_u = _fused_linear(xu, w_user, au_ext)
    z_item, s_i = _fused_linear(xi, w_item, ai_ext)

    c_u2i = jnp.zeros((_NP, _NP), f32).at[u2i[1], u2i[0]].add(1.0)
    c_i2u = jnp.zeros((_NP, _NP), f32).at[i2u[1], i2u[0]].add(1.0)

    h_item = _flash(c_u2i, s_i[:, _H:_H + _HP], s_u[:, :_HP].T, z_user)[:_N]
    h_user = _flash(c_i2u, s_u[:, _H:_H + _HP], s_i[:, :_HP].T, z_item)[:_N]
    return (h_user, h_item)
